# Initial kernel scaffold; baseline (speedup 1.0000x reference)
import jax
import jax.numpy as jnp
from jax.experimental import pallas as pl


def kernel(x, adj, W, a):
    raise NotImplementedError("write your pallas kernel here")


# speedup vs baseline: 2.6224x; 2.6224x over previous
import functools

import jax
import jax.numpy as jnp
from jax.experimental import pallas as pl
from jax.experimental.pallas import tpu as pltpu

ALPHA = 0.2


def _leaky(s):
    return jnp.where(s >= 0, s, ALPHA * s)


def _proj_kernel(nhid, x_ref, w_ref, a_ref, wh_ref, esrc_ref, edst_ref,
                 m_ref, meanwh_ref):
    wh = jnp.dot(x_ref[...], w_ref[...], preferred_element_type=jnp.float32)
    wh_ref[...] = wh
    a_all = a_ref[...]
    esrc = jnp.dot(wh, a_all[:nhid, :], preferred_element_type=jnp.float32)
    edst = jnp.dot(wh, a_all[nhid:, :], preferred_element_type=jnp.float32)
    esrc_ref[...] = esrc
    edst_ref[...] = edst
    m_ref[...] = _leaky(esrc + jnp.max(edst))
    meanwh_ref[...] = jnp.mean(wh, axis=0, keepdims=True)


def _flash_kernel(esrc_ref, m_ref, edstt_ref, adj_ref, wh_ref, meanwh_ref,
                  out_ref):
    s = _leaky(esrc_ref[...] + edstt_ref[...])
    p = adj_ref[...] * jnp.exp(s - m_ref[...])
    l = jnp.sum(p, axis=1, keepdims=True)
    h = jnp.dot(p, wh_ref[...], preferred_element_type=jnp.float32)
    h = jnp.where(l > 0, h / l, meanwh_ref[...])
    out_ref[...] = jnp.where(h > 0, h, jnp.exp(h) - 1.0)


def kernel(x, adj, W, a):
    n, _ = x.shape
    nhid = W.shape[1]
    f32 = jnp.float32

    wh, esrc, edst, m, meanwh = pl.pallas_call(
        functools.partial(_proj_kernel, nhid),
        out_shape=[
            jax.ShapeDtypeStruct((n, nhid), f32),
            jax.ShapeDtypeStruct((n, 1), f32),
            jax.ShapeDtypeStruct((n, 1), f32),
            jax.ShapeDtypeStruct((n, 1), f32),
            jax.ShapeDtypeStruct((1, nhid), f32),
        ],
    )(x, W, a)

    edst_t = edst.reshape(1, n)

    br = 200 if n % 200 == 0 else n
    num_rb = n // br

    out = pl.pallas_call(
        _flash_kernel,
        grid=(num_rb,),
        in_specs=[
            pl.BlockSpec((br, 1), lambda i: (i, 0)),
            pl.BlockSpec((br, 1), lambda i: (i, 0)),
            pl.BlockSpec((1, n), lambda i: (0, 0)),
            pl.BlockSpec((br, n), lambda i: (i, 0)),
            pl.BlockSpec((n, nhid), lambda i: (0, 0)),
            pl.BlockSpec((1, nhid), lambda i: (0, 0)),
        ],
        out_specs=pl.BlockSpec((br, nhid), lambda i: (i, 0)),
        out_shape=jax.ShapeDtypeStruct((n, nhid), f32),
        compiler_params=pltpu.CompilerParams(
            dimension_semantics=("arbitrary",),
        ),
    )(esrc, m, edst_t, adj, wh, meanwh)
    return out


# speedup vs baseline: 3.0940x; 1.1798x over previous
import functools

import jax
import jax.numpy as jnp
from jax.experimental import pallas as pl
from jax.experimental.pallas import tpu as pltpu

ALPHA = 0.2
LOG2E = 1.4426950408889634


def _proj_kernel(nhid, x_ref, w_ref, a_ref, whext_ref, negesrc_ref, u_ref,
                 v_ref, ep_ref, en_ref, meanwh_ref):
    wh = jnp.dot(x_ref[...], w_ref[...], preferred_element_type=jnp.float32)
    a_all = a_ref[...]
    esrc = jnp.dot(wh, a_all[:nhid, :], preferred_element_type=jnp.float32)
    edst = jnp.dot(wh, a_all[nhid:, :], preferred_element_type=jnp.float32)
    t = esrc + jnp.max(edst)
    m = jnp.where(t >= 0, t, ALPHA * t)
    negesrc_ref[...] = -esrc * LOG2E
    u_ref[...] = (esrc - m) * LOG2E
    v_ref[...] = (ALPHA * esrc - m) * LOG2E
    ep_ref[...] = edst * LOG2E
    en_ref[...] = (ALPHA * LOG2E) * edst
    whext_ref[:, :nhid] = wh
    whext_ref[:, nhid:] = jnp.ones_like(whext_ref[:, nhid:])
    meanwh_ref[...] = jnp.mean(wh, axis=0, keepdims=True)


def _flash_kernel(nhid, negesrc_ref, u_ref, v_ref, ept_ref, ent_ref, adj_ref,
                  whext_ref, meanwh_ref, out_ref):
    ept = ept_ref[...]
    mask = ept >= negesrc_ref[...]
    val = jnp.where(mask, u_ref[...] + ept, v_ref[...] + ent_ref[...])
    p = adj_ref[...] * jnp.exp2(val)
    h_ext = jnp.dot(p, whext_ref[...], preferred_element_type=jnp.float32)
    l = h_ext[:, nhid:nhid + 1]
    h = h_ext[:, :nhid]
    h = jnp.where(l > 0, h / l, meanwh_ref[...])
    out_ref[...] = jnp.where(h > 0, h, jnp.exp(h) - 1.0)


def kernel(x, adj, W, a):
    n, _ = x.shape
    nhid = W.shape[1]
    f32 = jnp.float32

    whext, negesrc, u, v, ep, en, meanwh = pl.pallas_call(
        functools.partial(_proj_kernel, nhid),
        out_shape=[
            jax.ShapeDtypeStruct((n, nhid + 1), f32),
            jax.ShapeDtypeStruct((n, 1), f32),
            jax.ShapeDtypeStruct((n, 1), f32),
            jax.ShapeDtypeStruct((n, 1), f32),
            jax.ShapeDtypeStruct((n, 1), f32),
            jax.ShapeDtypeStruct((n, 1), f32),
            jax.ShapeDtypeStruct((1, nhid), f32),
        ],
    )(x, W, a)

    ept = ep.reshape(1, n)
    ent = en.reshape(1, n)

    br = 200 if n % 200 == 0 else n
    num_rb = n // br

    out = pl.pallas_call(
        functools.partial(_flash_kernel, nhid),
        grid=(num_rb,),
        in_specs=[
            pl.BlockSpec((br, 1), lambda i: (i, 0)),
            pl.BlockSpec((br, 1), lambda i: (i, 0)),
            pl.BlockSpec((br, 1), lambda i: (i, 0)),
            pl.BlockSpec((1, n), lambda i: (0, 0)),
            pl.BlockSpec((1, n), lambda i: (0, 0)),
            pl.BlockSpec((br, n), lambda i: (i, 0)),
            pl.BlockSpec((n, nhid + 1), lambda i: (0, 0)),
            pl.BlockSpec((1, nhid), lambda i: (0, 0)),
        ],
        out_specs=pl.BlockSpec((br, nhid), lambda i: (i, 0)),
        out_shape=jax.ShapeDtypeStruct((n, nhid), f32),
        compiler_params=pltpu.CompilerParams(
            dimension_semantics=("arbitrary",),
        ),
    )(negesrc, u, v, ept, ent, adj, whext, meanwh)
    return out


# speedup vs baseline: 3.2113x; 1.0379x over previous
import functools

import jax
import jax.numpy as jnp
from jax.experimental import pallas as pl
from jax.experimental.pallas import tpu as pltpu

ALPHA = 0.2
LOG2E = 1.4426950408889634


def _proj_kernel(nhid, x_ref, w_ref, a_ref, whext_ref, u_ref,
                 v_ref, ep_ref, en_ref, meanwh_ref):
    wh = jnp.dot(x_ref[...], w_ref[...], preferred_element_type=jnp.float32)
    a_all = a_ref[...]
    esrc = jnp.dot(wh, a_all[:nhid, :], preferred_element_type=jnp.float32)
    edst = jnp.dot(wh, a_all[nhid:, :], preferred_element_type=jnp.float32)
    t = esrc + jnp.max(edst)
    m = jnp.where(t >= 0, t, ALPHA * t)
    u_ref[...] = (esrc - m) * LOG2E
    v_ref[...] = (ALPHA * esrc - m) * LOG2E
    ep_ref[...] = edst * LOG2E
    en_ref[...] = (ALPHA * LOG2E) * edst
    whext_ref[:, :nhid] = wh
    whext_ref[:, nhid:] = jnp.ones_like(whext_ref[:, nhid:])
    meanwh_ref[...] = jnp.mean(wh, axis=0, keepdims=True)


def _flash_kernel(nhid, u_ref, v_ref, ept_ref, ent_ref, adj_ref,
                  whext_ref, meanwh_ref, out_ref):
    val = jnp.maximum(u_ref[...] + ept_ref[...], v_ref[...] + ent_ref[...])
    p = adj_ref[...] * jnp.exp2(val)
    h_ext = jnp.dot(p, whext_ref[...], preferred_element_type=jnp.float32)
    l = h_ext[:, nhid:nhid + 1]
    h = h_ext[:, :nhid]
    h = jnp.where(l > 0, h / l, meanwh_ref[...])
    out_ref[...] = jnp.where(h > 0, h, jnp.exp(h) - 1.0)


def kernel(x, adj, W, a):
    n, _ = x.shape
    nhid = W.shape[1]
    f32 = jnp.float32

    whext, u, v, ep, en, meanwh = pl.pallas_call(
        functools.partial(_proj_kernel, nhid),
        out_shape=[
            jax.ShapeDtypeStruct((n, nhid + 1), f32),
            jax.ShapeDtypeStruct((n, 1), f32),
            jax.ShapeDtypeStruct((n, 1), f32),
            jax.ShapeDtypeStruct((n, 1), f32),
            jax.ShapeDtypeStruct((n, 1), f32),
            jax.ShapeDtypeStruct((1, nhid), f32),
        ],
    )(x, W, a)

    ept = ep.reshape(1, n)
    ent = en.reshape(1, n)

    br = 200 if n % 200 == 0 else n
    num_rb = n // br

    out = pl.pallas_call(
        functools.partial(_flash_kernel, nhid),
        grid=(num_rb,),
        in_specs=[
            pl.BlockSpec((br, 1), lambda i: (i, 0)),
            pl.BlockSpec((br, 1), lambda i: (i, 0)),
            pl.BlockSpec((1, n), lambda i: (0, 0)),
            pl.BlockSpec((1, n), lambda i: (0, 0)),
            pl.BlockSpec((br, n), lambda i: (i, 0)),
            pl.BlockSpec((n, nhid + 1), lambda i: (0, 0)),
            pl.BlockSpec((1, nhid), lambda i: (0, 0)),
        ],
        out_specs=pl.BlockSpec((br, nhid), lambda i: (i, 0)),
        out_shape=jax.ShapeDtypeStruct((n, nhid), f32),
        compiler_params=pltpu.CompilerParams(
            dimension_semantics=("arbitrary",),
        ),
    )(u, v, ept, ent, adj, whext, meanwh)
    return out


# speedup vs baseline: 3.3707x; 1.0496x over previous
import functools

import jax
import jax.numpy as jnp
from jax.experimental import pallas as pl
from jax.experimental.pallas import tpu as pltpu

ALPHA = 0.2
LOG2E = 1.4426950408889634


def _proj_kernel(nhid, x_ref, w_ref, a_ref, whext_ref, u_ref,
                 v_ref, ep_ref, en_ref, meanwh_ref):
    wh = jnp.dot(x_ref[...], w_ref[...], preferred_element_type=jnp.float32)
    a_all = a_ref[...]
    esrc = jnp.dot(wh, a_all[:nhid, :], preferred_element_type=jnp.float32)
    edst = jnp.dot(wh, a_all[nhid:, :], preferred_element_type=jnp.float32)
    t = esrc + jnp.max(edst)
    m = jnp.where(t >= 0, t, ALPHA * t)
    u_ref[...] = (esrc - m) * LOG2E
    v_ref[...] = (ALPHA * esrc - m) * LOG2E
    ep_ref[...] = edst * LOG2E
    en_ref[...] = (ALPHA * LOG2E) * edst
    whext_ref[:, :nhid] = wh
    whext_ref[:, nhid:] = jnp.ones_like(whext_ref[:, nhid:])
    meanwh_ref[...] = jnp.mean(wh, axis=0, keepdims=True)


def _flash_kernel(nhid, u_ref, v_ref, ept_ref, ent_ref, adj_ref,
                  whext_ref, meanwh_ref, out_ref):
    val = jnp.maximum(u_ref[...] + ept_ref[...], v_ref[...] + ent_ref[...])
    p = adj_ref[...] * jnp.exp2(val)
    h_ext = jnp.dot(p, whext_ref[...], preferred_element_type=jnp.float32)
    l = h_ext[:, nhid:nhid + 1]
    h = h_ext[:, :nhid]
    h = jnp.where(l > 0, h / l, meanwh_ref[...])
    out_ref[...] = jnp.where(h > 0, h, jnp.exp(h) - 1.0)


def kernel(x, adj, W, a):
    n, _ = x.shape
    nhid = W.shape[1]
    f32 = jnp.float32

    whext, u, v, ep, en, meanwh = pl.pallas_call(
        functools.partial(_proj_kernel, nhid),
        out_shape=[
            jax.ShapeDtypeStruct((n, nhid + 1), f32),
            jax.ShapeDtypeStruct((n, 1), f32),
            jax.ShapeDtypeStruct((n, 1), f32),
            jax.ShapeDtypeStruct((n, 1), f32),
            jax.ShapeDtypeStruct((n, 1), f32),
            jax.ShapeDtypeStruct((1, nhid), f32),
        ],
    )(x, W, a)

    ept = ep.reshape(1, n)
    ent = en.reshape(1, n)

    br = 400 if n % 400 == 0 else n
    num_rb = n // br

    out = pl.pallas_call(
        functools.partial(_flash_kernel, nhid),
        grid=(num_rb,),
        in_specs=[
            pl.BlockSpec((br, 1), lambda i: (i, 0)),
            pl.BlockSpec((br, 1), lambda i: (i, 0)),
            pl.BlockSpec((1, n), lambda i: (0, 0)),
            pl.BlockSpec((1, n), lambda i: (0, 0)),
            pl.BlockSpec((br, n), lambda i: (i, 0)),
            pl.BlockSpec((n, nhid + 1), lambda i: (0, 0)),
            pl.BlockSpec((1, nhid), lambda i: (0, 0)),
        ],
        out_specs=pl.BlockSpec((br, nhid), lambda i: (i, 0)),
        out_shape=jax.ShapeDtypeStruct((n, nhid), f32),
        compiler_params=pltpu.CompilerParams(
            dimension_semantics=("arbitrary",),
        ),
    )(u, v, ept, ent, adj, whext, meanwh)
    return out
